# baseline (speedup 1.0000x reference)
import jax
import jax.numpy as jnp
from jax.experimental import pallas as pl


def _pos_kernel(row_t_ref, col_t_ref, out_ref):
    k = pl.program_id(1)

    @pl.when(k == 0)
    def _():
        r = row_t_ref[...]
        out_ref[...] = jnp.broadcast_to(
            r[None, :, :, None], out_ref.shape
        )

    @pl.when(k == 1)
    def _():
        c = col_t_ref[...]
        out_ref[...] = jnp.broadcast_to(
            c[None, :, None, :], out_ref.shape
        )


def kernel(x, row_embed, col_embed):
    b = x.shape[0]
    h, w = x.shape[-2], x.shape[-1]
    d = row_embed.shape[-1]
    row_t = row_embed.T
    col_t = col_embed.T
    return pl.pallas_call(
        _pos_kernel,
        grid=(b, 2),
        in_specs=[
            pl.BlockSpec((d, h), lambda bb, k: (0, 0)),
            pl.BlockSpec((d, w), lambda bb, k: (0, 0)),
        ],
        out_specs=pl.BlockSpec((1, d, h, w), lambda bb, k: (bb, k, 0, 0)),
        out_shape=jax.ShapeDtypeStruct((b, 2 * d, h, w), x.dtype),
    )(row_t, col_t)
